# baseline (speedup 1.0000x reference)
import functools
import math

import jax
import jax.numpy as jnp
import numpy as np
from jax import lax
from jax.experimental import pallas as pl
from jax.experimental.pallas import tpu as pltpu
from jax.experimental.pallas import tpu_sc as plsc

NC = 2
NS = 16
NW = NC * NS
LANES = 16
CH = 16
TB = 4
EPS = 1e-5


def _rsqrt(x):
    bits = lax.bitcast_convert_type(x, jnp.int32)
    y = lax.bitcast_convert_type(jnp.int32(0x5F3759DF) - (bits >> 1),
                                 jnp.float32)
    for _ in range(3):
        y = y * (1.5 - 0.5 * x * y * y)
    return y


@functools.lru_cache(maxsize=None)
def _positional_table(seq_len, d_b):
    pos = np.arange(seq_len, dtype=np.float32)[:, None]
    div = np.exp(
        np.arange(0, d_b, 2, dtype=np.float32) * (-(math.log(10000.0) / d_b)))
    pe = np.zeros((seq_len, d_b), dtype=np.float32)
    pe[:, 0::2] = np.sin(pos * div)
    pe[:, 1::2] = np.cos(pos * div[: d_b // 2 + d_b % 2])
    return jnp.asarray(pe)


@functools.lru_cache(maxsize=None)
def _build(n_tok, seq_len, d):
    dm = 3 * d
    tok_w = n_tok // NW
    nch = tok_w // CH
    ncv = d // LANES

    mesh = plsc.VectorSubcoreMesh(core_axis_name="c", subcore_axis_name="s")

    buf_t = pltpu.VMEM((CH, d), jnp.float32)

    @functools.partial(
        pl.kernel,
        out_type=jax.ShapeDtypeStruct((n_tok, dm), jnp.float32),
        mesh=mesh,
        compiler_params=pltpu.CompilerParams(needs_layout_passes=False),
        scratch_types=[
            pltpu.VMEM((tok_w,), jnp.int32),
            [buf_t, buf_t],
            [buf_t, buf_t],
            [buf_t, buf_t],
            [buf_t, buf_t],
            [pltpu.VMEM((CH, dm), jnp.float32)] * 2,
            pltpu.VMEM((dm,), jnp.float32),
            pltpu.VMEM((dm,), jnp.float32),
            pltpu.VMEM((CH,), jnp.float32),
            pltpu.VMEM((CH,), jnp.float32),
            [pltpu.SemaphoreType.DMA] * 2,
            [pltpu.SemaphoreType.DMA] * 2,
        ],
    )
    def launch(tok_hbm, wa_hbm, wf_hbm, wb_hbm, gamma_hbm, beta_hbm, pe_hbm,
               out_hbm, tid_v, a_v, f_v, b_v, pe_v, o_v, g_v, bb_v,
               mu_v, rs_v, gsem, osem):
        wid = lax.axis_index("s") * NC + lax.axis_index("c")
        base = wid * tok_w

        pltpu.sync_copy(tok_hbm.at[pl.ds(base, tok_w)], tid_v)
        pltpu.sync_copy(gamma_hbm, g_v)
        pltpu.sync_copy(beta_hbm, bb_v)

        def issue(c, s):
            idx = tid_v.at[pl.ds(c * CH, CH)]
            pos = lax.rem(base + c * CH, seq_len)
            pltpu.async_copy(wa_hbm.at[idx], a_v[s], gsem[s])
            pltpu.async_copy(wf_hbm.at[idx], f_v[s], gsem[s])
            pltpu.async_copy(wb_hbm.at[idx], b_v[s], gsem[s])
            pltpu.async_copy(pe_hbm.at[pl.ds(pos, CH)], pe_v[s], gsem[s])

        def wait_in(s):
            for hbm in (wa_hbm, wf_hbm, wb_hbm):
                pltpu.make_async_copy(
                    hbm.at[pl.ds(0, CH)], a_v[s], gsem[s]).wait()
            pltpu.make_async_copy(
                pe_hbm.at[pl.ds(0, CH)], pe_v[s], gsem[s]).wait()

        def wait_out(s):
            pltpu.make_async_copy(
                o_v[s], out_hbm.at[pl.ds(0, CH)], osem[s]).wait()

        def compute(s):
            bufs = (a_v[s], f_v[s], b_v[s])
            lane = lax.iota(jnp.int32, LANES)

            def p1(col, carry):
                s0, q0 = carry
                cols = jnp.full((LANES,), col, jnp.int32)
                xa = plsc.load_gather(a_v[s], [lane, cols])
                xf = plsc.load_gather(f_v[s], [lane, cols])
                xb = (plsc.load_gather(b_v[s], [lane, cols])
                      + plsc.load_gather(pe_v[s], [lane, cols]))
                s0 = s0 + xa + xf + xb
                q0 = q0 + xa * xa + xf * xf + xb * xb
                return s0, q0

            s0, q0 = lax.fori_loop(
                0, d, p1,
                (jnp.zeros((LANES,), jnp.float32),
                 jnp.zeros((LANES,), jnp.float32)))
            mu16 = s0 * (1.0 / dm)
            var16 = q0 * (1.0 / dm) - mu16 * mu16
            mu_v[pl.ds(0, LANES)] = mu16
            rs_v[pl.ds(0, LANES)] = _rsqrt(var16 + EPS)

            def p2j(j, blk):
                t0 = blk * TB
                ms = [plsc.load_gather(
                    mu_v, [jnp.full((LANES,), t0 + i, jnp.int32)])
                    for i in range(TB)]
                rs = [plsc.load_gather(
                    rs_v, [jnp.full((LANES,), t0 + i, jnp.int32)])
                    for i in range(TB)]
                for sec, buf in enumerate(bufs):
                    col = sec * d + j * LANES
                    g = g_v[pl.ds(col, LANES)]
                    bb = bb_v[pl.ds(col, LANES)]
                    for i in range(TB):
                        x = buf[t0 + i, pl.ds(j * LANES, LANES)]
                        if sec == 2:
                            x = x + pe_v[s][t0 + i, pl.ds(j * LANES, LANES)]
                        gs = g * rs[i]
                        o_v[s][t0 + i, pl.ds(col, LANES)] = (
                            x * gs + (bb - ms[i] * gs))
                return blk

            def p2(blk, _):
                lax.fori_loop(0, ncv, p2j, blk)
                return 0

            lax.fori_loop(0, CH // TB, p2, 0)

        def phase(c, s, g, first, last):
            wait_in(s)
            if last is None:
                issue(c + 1, 1 - s)
            else:
                @pl.when(jnp.logical_not(last))
                def _():
                    issue(c + 1, 1 - s)
            if first is not None:
                @pl.when(jnp.logical_not(first))
                def _():
                    wait_out(s)
            compute(s)
            pltpu.async_copy(
                o_v[s], out_hbm.at[pl.ds(base + c * CH, CH)], osem[s])

        issue(0, 0)

        def body(g, _):
            c0 = 2 * g
            phase(c0, 0, g, first=(g == 0), last=None)
            phase(c0 + 1, 1, g, first=(g == 0), last=(g == nch // 2 - 1))
            return 0

        lax.fori_loop(0, nch // 2, body, 0)
        wait_out(0)
        wait_out(1)

    return launch


def kernel(tokens, Wa, Wf, Wb, gamma, beta):
    b, s = tokens.shape
    d = Wa.shape[1]
    tok = tokens.reshape(-1).astype(jnp.int32)
    pe = _positional_table(s, Wb.shape[1])
    out = _build(b * s, s, d)(tok, Wa, Wf, Wb, gamma, beta, pe)
    return out.reshape(b, s, 3 * d)


# speedup vs baseline: 3.8481x; 3.8481x over previous
import functools
import math

import jax
import jax.numpy as jnp
import numpy as np
from jax import lax
from jax.experimental import pallas as pl
from jax.experimental.pallas import tpu as pltpu
from jax.experimental.pallas import tpu_sc as plsc

NC = 2
NS = 16
NW = NC * NS
LANES = 16
CH = 8
TB = 4
EPS = 1e-5


def _rsqrt(x):
    bits = lax.bitcast_convert_type(x, jnp.int32)
    y = lax.bitcast_convert_type(jnp.int32(0x5F3759DF) - (bits >> 1),
                                 jnp.float32)
    for _ in range(3):
        y = y * (1.5 - 0.5 * x * y * y)
    return y


_GATHER_DNUMS = lax.GatherDimensionNumbers(
    offset_dims=(), collapsed_slice_dims=(0,), start_index_map=(0,))


def _shuffle(x, idx):
    return lax.gather(x, idx[:, None], _GATHER_DNUMS, (1,),
                      mode=lax.GatherScatterMode.PROMISE_IN_BOUNDS)


def _lane_sum(x):
    idx = lax.iota(jnp.int32, LANES)
    for k in (1, 2, 4, 8):
        x = x + _shuffle(x, idx ^ k)
    return x


@functools.lru_cache(maxsize=None)
def _positional_table(seq_len, d_b):
    pos = np.arange(seq_len, dtype=np.float32)[:, None]
    div = np.exp(
        np.arange(0, d_b, 2, dtype=np.float32) * (-(math.log(10000.0) / d_b)))
    pe = np.zeros((seq_len, d_b), dtype=np.float32)
    pe[:, 0::2] = np.sin(pos * div)
    pe[:, 1::2] = np.cos(pos * div[: d_b // 2 + d_b % 2])
    return jnp.asarray(pe)


@functools.lru_cache(maxsize=None)
def _build(n_tok, seq_len, d):
    dm = 3 * d
    tok_w = n_tok // NW
    nch = tok_w // CH
    ncv = d // LANES

    mesh = plsc.VectorSubcoreMesh(core_axis_name="c", subcore_axis_name="s")

    buf_t = pltpu.VMEM((CH, d), jnp.float32)

    @functools.partial(
        pl.kernel,
        out_type=jax.ShapeDtypeStruct((n_tok, dm), jnp.float32),
        mesh=mesh,
        compiler_params=pltpu.CompilerParams(needs_layout_passes=False),
        scratch_types=[
            pltpu.VMEM((tok_w,), jnp.int32),
            [buf_t, buf_t],
            [buf_t, buf_t],
            [buf_t, buf_t],
            [buf_t, buf_t],
            [pltpu.VMEM((CH, dm), jnp.float32)] * 2,
            pltpu.VMEM((dm,), jnp.float32),
            pltpu.VMEM((dm,), jnp.float32),
            [pltpu.SemaphoreType.DMA] * 2,
            [pltpu.SemaphoreType.DMA] * 2,
        ],
    )
    def launch(tok_hbm, wa_hbm, wf_hbm, wb_hbm, gamma_hbm, beta_hbm, pe_hbm,
               out_hbm, tid_v, a_v, f_v, b_v, pe_v, o_v, g_v, bb_v,
               gsem, osem):
        wid = lax.axis_index("s") * NC + lax.axis_index("c")
        base = wid * tok_w

        pltpu.sync_copy(tok_hbm.at[pl.ds(base, tok_w)], tid_v)
        pltpu.sync_copy(gamma_hbm, g_v)
        pltpu.sync_copy(beta_hbm, bb_v)

        def issue(c, s):
            idx = tid_v.at[pl.ds(c * CH, CH)]
            pos = lax.rem(base + c * CH, seq_len)
            pltpu.async_copy(wa_hbm.at[idx], a_v[s], gsem[s])
            pltpu.async_copy(wf_hbm.at[idx], f_v[s], gsem[s])
            pltpu.async_copy(wb_hbm.at[idx], b_v[s], gsem[s])
            pltpu.async_copy(pe_hbm.at[pl.ds(pos, CH)], pe_v[s], gsem[s])

        def wait_in(s):
            for hbm in (wa_hbm, wf_hbm, wb_hbm):
                pltpu.make_async_copy(
                    hbm.at[pl.ds(0, CH)], a_v[s], gsem[s]).wait()
            pltpu.make_async_copy(
                pe_hbm.at[pl.ds(0, CH)], pe_v[s], gsem[s]).wait()

        def wait_out(s):
            pltpu.make_async_copy(
                o_v[s], out_hbm.at[pl.ds(0, CH)], osem[s]).wait()

        def compute(s):
            bufs = (a_v[s], f_v[s], b_v[s])

            def p1(j, carry):
                sv = list(carry[:CH])
                qv = list(carry[CH:])
                for sec, buf in enumerate(bufs):
                    for t in range(CH):
                        x = buf[t, pl.ds(j * LANES, LANES)]
                        if sec == 2:
                            x = x + pe_v[s][t, pl.ds(j * LANES, LANES)]
                        sv[t] = sv[t] + x
                        qv[t] = qv[t] + x * x
                return tuple(sv) + tuple(qv)

            zero = jnp.zeros((LANES,), jnp.float32)
            acc = lax.fori_loop(0, ncv, p1, (zero,) * (2 * CH))

            ms = []
            rs = []
            for t in range(CH):
                mu = _lane_sum(acc[t]) * (1.0 / dm)
                var = _lane_sum(acc[CH + t]) * (1.0 / dm) - mu * mu
                ms.append(mu)
                rs.append(_rsqrt(var + EPS))

            def p2(j, carry):
                cms = carry[:CH]
                crs = carry[CH:]
                gl = []
                bl = []
                xs = []
                for sec in range(3):
                    col = sec * d + j * LANES
                    gl.append(g_v[pl.ds(col, LANES)])
                    bl.append(bb_v[pl.ds(col, LANES)])
                for sec, buf in enumerate(bufs):
                    row = []
                    for t in range(CH):
                        x = buf[t, pl.ds(j * LANES, LANES)]
                        if sec == 2:
                            x = x + pe_v[s][t, pl.ds(j * LANES, LANES)]
                        row.append(x)
                    xs.append(row)
                for sec in range(3):
                    col = sec * d + j * LANES
                    for t in range(CH):
                        z = (xs[sec][t] - cms[t]) * crs[t]
                        o_v[s][t, pl.ds(col, LANES)] = z * gl[sec] + bl[sec]
                return carry

            lax.fori_loop(0, ncv, p2, tuple(ms) + tuple(rs))

        def phase(c, s, g, first, last):
            wait_in(s)
            if last is None:
                issue(c + 1, 1 - s)
            else:
                @pl.when(jnp.logical_not(last))
                def _():
                    issue(c + 1, 1 - s)
            if first is not None:
                @pl.when(jnp.logical_not(first))
                def _():
                    wait_out(s)
            compute(s)
            pltpu.async_copy(
                o_v[s], out_hbm.at[pl.ds(base + c * CH, CH)], osem[s])

        issue(0, 0)

        def body(g, _):
            c0 = 2 * g
            phase(c0, 0, g, first=(g == 0), last=None)
            phase(c0 + 1, 1, g, first=(g == 0), last=(g == nch // 2 - 1))
            return 0

        lax.fori_loop(0, nch // 2, body, 0)
        wait_out(0)
        wait_out(1)

    return launch


def kernel(tokens, Wa, Wf, Wb, gamma, beta):
    b, s = tokens.shape
    d = Wa.shape[1]
    tok = tokens.reshape(-1).astype(jnp.int32)
    pe = _positional_table(s, Wb.shape[1])
    out = _build(b * s, s, d)(tok, Wa, Wf, Wb, gamma, beta, pe)
    return out.reshape(b, s, 3 * d)
